# baseline (speedup 1.0000x reference)
import jax
import jax.numpy as jnp
import numpy as np
from jax.experimental import pallas as pl


def _gcn_conv(x, W, b, src, dst, n):
    h = x @ W
    loop = jnp.arange(n, dtype=src.dtype)
    s = jnp.concatenate([src, loop])
    d = jnp.concatenate([dst, loop])
    deg = jnp.zeros((n,), dtype=h.dtype).at[d].add(1.0)
    dinv = 1.0 / jnp.sqrt(jnp.maximum(deg, 1.0))
    norm = (dinv[s] * dinv[d])[:, None]
    out = jnp.zeros_like(h).at[d].add(h[s] * norm)
    return out + b


def _proj_kernel(ev_ref, w_ref, b_ref, o_ref):
    o_ref[...] = (ev_ref[...] @ w_ref[...] + b_ref[...])[..., 0]


def kernel(x, edge_index, W1, b1, W2, b2, W3, b3, Wa1, ba1, Wa2, ba2, W_ih, b_ih, W_hh, b_hh, Wo1, bo1, Wo2, bo2, Wout, bout):
    B, N, T = x.shape
    src, dst = edge_index[0], edge_index[1]
    n = B * N
    H = W_hh.shape[1]
    FORECAST = 6
    hs = []
    for t in range(T):
        h = x[:, :, t].reshape(n, 1)
        h = jax.nn.relu(_gcn_conv(h, W1, b1, src, dst, n))
        h = jax.nn.relu(_gcn_conv(h, W2, b2, src, dst, n))
        h = jax.nn.relu(_gcn_conv(h, W3, b3, src, dst, n))
        hs.append(h.reshape(B, N, -1))
    h_seq = jnp.stack(hs, axis=1)
    att = jnp.tanh(h_seq @ Wa1 + ba1) @ Wa2 + ba2
    w = jax.nn.softmax(att, axis=1)
    nf = jnp.sum(h_seq * w, axis=1).reshape(n, -1)
    gx = nf @ W_ih.T + b_ih
    gh = jnp.zeros((n, H), nf.dtype) @ W_hh.T + b_hh
    r = jax.nn.sigmoid(gx[:, :H] + gh[:, :H])
    z = jax.nn.sigmoid(gx[:, H:2 * H] + gh[:, H:2 * H])
    cand = jnp.tanh(gx[:, 2 * H:] + r * gh[:, 2 * H:])
    hidden = (1.0 - z) * cand

    def f(h):
        h = jnp.tanh(_gcn_conv(h, Wo1, bo1, src, dst, n))
        h = jnp.tanh(_gcn_conv(h, Wo2, bo2, src, dst, n))
        return h

    ts = np.linspace(0.0, FORECAST, FORECAST)
    ys = [hidden]
    y = hidden
    for i in range(FORECAST - 1):
        dt = float(ts[i + 1] - ts[i])
        k1 = f(y)
        k2 = f(y + dt * 0.5 * k1)
        k3 = f(y + dt * 0.5 * k2)
        k4 = f(y + dt * k3)
        y = y + (dt / 6.0) * (k1 + 2.0 * k2 + 2.0 * k3 + k4)
        ys.append(y)
    ev = jnp.stack(ys, 0).reshape(FORECAST, B, N, H).transpose(1, 2, 0, 3)

    pred = pl.pallas_call(
        _proj_kernel,
        out_shape=jax.ShapeDtypeStruct(ev.shape[:-1], ev.dtype),
    )(ev, Wout, bout)
    return pred


# speedup vs baseline: 25.6154x; 25.6154x over previous
import functools

import jax
import jax.numpy as jnp
from jax import lax
from jax.experimental import pallas as pl
from jax.experimental.pallas import tpu as pltpu
from jax.experimental.pallas import tpu_sc as plsc

N_NODES = 10000
CHUNK = 80
NW = 32
F32 = jnp.float32


def _build_spmm(K: int, W: int, E: int):
    n = N_NODES
    epw = E // NW
    nchunk = epw // CHUNK
    rows_per_chunkrow = CHUNK
    cpw = epw // rows_per_chunkrow

    mesh = plsc.VectorSubcoreMesh(core_axis_name="c", subcore_axis_name="s",
                                  num_cores=2, num_subcores=16)

    def body(g_hbm, edges_hbm, out_hbm, src2, dst2, idxp, stage, zbuf, acc, sem):
        c = lax.axis_index("c")
        s = lax.axis_index("s")
        wid = s * 2 + c

        pltpu.sync_copy(edges_hbm.at[0, wid], src2)
        pltpu.sync_copy(edges_hbm.at[1, wid], dst2)

        def zrow(r, _):
            for i in range(W // 16):
                zbuf[r, pl.ds(i * 16, 16)] = jnp.zeros((16,), F32)
            return 0
        lax.fori_loop(0, 80, zrow, 0)

        def zero_stripe():
            @pl.when(s != 15)
            def _():
                def zm(m, _):
                    pltpu.sync_copy(zbuf, acc.at[pl.ds(s * 640 + m * 80, 80)])
                    return 0
                lax.fori_loop(0, 8, zm, 0)

            @pl.when(s == 15)
            def _():
                def zm(m, _):
                    pltpu.sync_copy(zbuf, acc.at[pl.ds(9600 + m * 80, 80)])
                    return 0
                lax.fori_loop(0, 5, zm, 0)

        zero_stripe()
        plsc.subcore_barrier()

        def fire(j, slot, koff):
            for i in range(CHUNK // 16):
                idxp[slot, pl.ds(i * 16, 16)] = src2[j, pl.ds(i * 16, 16)] + koff
            pltpu.make_async_copy(g_hbm.at[idxp.at[slot]], stage.at[slot], sem).start()

        def kbody(k, _):
            koff = k * n
            fire(0, 0, koff)

            def cbody(j, _):
                slot = lax.rem(j, 2)

                @pl.when(j + 1 < nchunk)
                def _():
                    fire(j + 1, lax.rem(j + 1, 2), koff)

                pltpu.make_async_copy(g_hbm.at[idxp.at[slot]], stage.at[slot], sem).wait()
                pltpu.sync_copy(stage.at[slot], acc.at[dst2.at[j]], add=True)
                return 0

            lax.fori_loop(0, nchunk, cbody, 0)
            plsc.subcore_barrier()

            obase = (c * K + k) * n

            @pl.when(s != 15)
            def _():
                pltpu.sync_copy(acc.at[pl.ds(s * 640, 640)],
                                out_hbm.at[pl.ds(obase + s * 640, 640)])

            @pl.when(s == 15)
            def _():
                pltpu.sync_copy(acc.at[pl.ds(9600, 400)],
                                out_hbm.at[pl.ds(obase + 9600, 400)])

            zero_stripe()
            plsc.subcore_barrier()
            return 0

        lax.fori_loop(0, K, kbody, 0)

    kern = pl.kernel(
        body,
        out_type=jax.ShapeDtypeStruct((2 * K * n, W), F32),
        mesh=mesh,
        compiler_params=pltpu.CompilerParams(use_tc_tiling_on_sc=False),
        scratch_types=[
            pltpu.VMEM((cpw, CHUNK), jnp.int32),
            pltpu.VMEM((cpw, CHUNK), jnp.int32),
            pltpu.VMEM((2, CHUNK), jnp.int32),
            pltpu.VMEM((2, CHUNK, W), F32),
            pltpu.VMEM((80, W), F32),
            pltpu.VMEM_SHARED((n, W), F32),
            pltpu.SemaphoreType.DMA,
        ],
    )
    return kern


BN = 1000


def _full(shape):
    return pl.BlockSpec(shape, lambda i: tuple(0 for _ in shape))


def _tc_call(body, in_specs, out_specs, out_shapes):
    return pl.pallas_call(
        body,
        grid=(N_NODES // BN,),
        in_specs=in_specs,
        out_specs=out_specs,
        out_shape=out_shapes,
    )


def _nb(*lead):
    def mk(trail):
        shape = tuple(lead) + (BN,) + tuple(trail)
        nlead = len(lead)
        def imap(i):
            return tuple(0 for _ in lead) + (i,) + tuple(0 for _ in trail)
        return pl.BlockSpec(shape, imap)
    return mk


def _tc1_body(deg2, xpad, dinv64, xg):
    deg = deg2[0, :, 0] + deg2[1, :, 0] + 1.0
    dinv = lax.rsqrt(deg)
    dinv64[...] = jnp.broadcast_to(dinv[:, None], dinv64.shape)
    xg[...] = xpad[...] * dinv[:, None]


def _tc2_body(s1, xg, dinv64, W1, b1, W2, zg2):
    dv = dinv64[...]
    Y = (s1[0] + s1[1] + xg[...]) * dv[:, :16]
    w1 = W1[...][0, :]
    for t in range(12):
        h1 = jnp.maximum(Y[:, t][:, None] * w1[None, :] + b1[...][None, :], 0.0)
        zg2[t] = jnp.dot(h1, W2[...], preferred_element_type=F32) * dv


def _tc3_body(sp, zg, dinv64, b, Wn, zgn):
    dv = dinv64[...]
    for t in range(12):
        h = jnp.maximum((sp[0, t] + sp[1, t] + zg[t]) * dv + b[...][None, :], 0.0)
        zgn[t] = jnp.dot(h, Wn[...], preferred_element_type=F32) * dv


def _tc4_body(sp, zg, dinv64, b3, Wa1, ba1, Wa2, ba2, WihT, bih, bhh, Wo1,
              hid_out, ug_out):
    dv = dinv64[...]
    hs = []
    ats = []
    for t in range(12):
        h = jnp.maximum((sp[0, t] + sp[1, t] + zg[t]) * dv + b3[...][None, :], 0.0)
        hs.append(h)
        a = jnp.tanh(jnp.dot(h, Wa1[...], preferred_element_type=F32) + ba1[...][None, :])
        ats.append(jnp.dot(a, Wa2[...], preferred_element_type=F32) + ba2[...][None, :])
    att = jnp.concatenate(ats, axis=1)
    m = jnp.max(att, axis=1, keepdims=True)
    e = jnp.exp(att - m)
    w = e / jnp.sum(e, axis=1, keepdims=True)
    nf = hs[0] * w[:, 0][:, None]
    for t in range(1, 12):
        nf = nf + hs[t] * w[:, t][:, None]
    gx = jnp.dot(nf, WihT[...], preferred_element_type=F32) + bih[...][None, :]
    bh = bhh[...]
    r = jax.nn.sigmoid(gx[:, :64] + bh[None, :64])
    z = jax.nn.sigmoid(gx[:, 64:128] + bh[None, 64:128])
    cand = jnp.tanh(gx[:, 128:] + r * bh[None, 128:])
    hidden = (1.0 - z) * cand
    hid_out[...] = hidden
    ug_out[...] = jnp.dot(hidden, Wo1[...], preferred_element_type=F32) * dv


def _tc5_body(sp, ug, dinv64, bo1, Wo2, vg):
    dv = dinv64[...]
    q = jnp.tanh((sp[0] + sp[1] + ug[...]) * dv + bo1[...][None, :])
    vg[...] = jnp.dot(q, Wo2[...], preferred_element_type=F32) * dv


def _tc6_body(sp, vg, dinv64, bo2, y, kacc, Wo1, kacc_out, ynext, ugnext,
              *, init, final, alpha, beta):
    dv = dinv64[...]
    kk = jnp.tanh((sp[0] + sp[1] + vg[...]) * dv + bo2[...][None, :])
    ka = kk if init else kacc[...] + beta * kk
    kacc_out[...] = ka
    yn = y[...] + (alpha * ka if final else alpha * kk)
    ynext[...] = yn
    ugnext[...] = jnp.dot(yn, Wo1[...], preferred_element_type=F32) * dv


def _tc7_body(ys, Wout, bout, out):
    cols = []
    for i in range(6):
        cols.append(jnp.dot(ys[i], Wout[...], preferred_element_type=F32)
                    + bout[...][None, :])
    out[...] = jnp.concatenate(cols, axis=1)[None]


def kernel(x, edge_index, W1, b1, W2, b2, W3, b3, Wa1, ba1, Wa2, ba2,
           W_ih, b_ih, W_hh, b_hh, Wo1, bo1, Wo2, bo2, Wout, bout):
    B, N, T = x.shape
    n = B * N
    E = edge_index.shape[1]
    edges_r = edge_index.reshape(2, NW, E // (NW * CHUNK), CHUNK)

    spmm16 = _build_spmm(1, 16, E)
    spmm64_12 = _build_spmm(12, 64, E)
    spmm64_1 = _build_spmm(1, 64, E)

    nb = _nb()
    nb2 = _nb(2)
    nb12 = _nb(12)
    nb2_12 = _nb(2, 12)
    nb6 = _nb(6)
    nb1 = _nb(1)

    ones16 = jnp.ones((n, 16), F32)
    deg2 = spmm16(ones16, edges_r).reshape(2, n, 16)

    xpad = jnp.concatenate([x.reshape(n, T), jnp.zeros((n, 16 - T), F32)], axis=1)

    dinv64, xg = _tc_call(
        _tc1_body,
        [nb2((16,)), nb((16,))],
        [nb((64,)), nb((16,))],
        [jax.ShapeDtypeStruct((n, 64), F32), jax.ShapeDtypeStruct((n, 16), F32)],
    )(deg2, xpad)

    s1 = spmm16(xg, edges_r).reshape(2, n, 16)

    zg2 = _tc_call(
        _tc2_body,
        [nb2((16,)), nb((16,)), nb((64,)), _full((1, 64)), _full((64,)), _full((64, 64))],
        nb12((64,)),
        jax.ShapeDtypeStruct((12, n, 64), F32),
    )(s1, xg, dinv64, W1, b1, W2)

    s2 = spmm64_12(zg2.reshape(12 * n, 64), edges_r).reshape(2, 12, n, 64)

    zg3 = _tc_call(
        _tc3_body,
        [nb2_12((64,)), nb12((64,)), nb((64,)), _full((64,)), _full((64, 64))],
        nb12((64,)),
        jax.ShapeDtypeStruct((12, n, 64), F32),
    )(s2, zg2, dinv64, b2, W3)

    s3 = spmm64_12(zg3.reshape(12 * n, 64), edges_r).reshape(2, 12, n, 64)

    hidden, ug = _tc_call(
        _tc4_body,
        [nb2_12((64,)), nb12((64,)), nb((64,)), _full((64,)),
         _full((64, 64)), _full((64,)), _full((64, 1)), _full((1,)),
         _full((64, 192)), _full((192,)), _full((192,)), _full((64, 64))],
        [nb((64,)), nb((64,))],
        [jax.ShapeDtypeStruct((n, 64), F32), jax.ShapeDtypeStruct((n, 64), F32)],
    )(s3, zg3, dinv64, b3, Wa1, ba1, Wa2, ba2, W_ih.T, b_ih, b_hh, Wo1)

    tc5 = _tc_call(
        _tc5_body,
        [nb2((64,)), nb((64,)), nb((64,)), _full((64,)), _full((64, 64))],
        nb((64,)),
        jax.ShapeDtypeStruct((n, 64), F32),
    )

    def tc6(init, final, alpha, beta):
        return _tc_call(
            functools.partial(_tc6_body, init=init, final=final,
                              alpha=alpha, beta=beta),
            [nb2((64,)), nb((64,)), nb((64,)), _full((64,)),
             nb((64,)), nb((64,)), _full((64, 64))],
            [nb((64,)), nb((64,)), nb((64,))],
            [jax.ShapeDtypeStruct((n, 64), F32)] * 3,
        )

    dt = 6.0 / 5.0
    stages = [
        tc6(True, False, 0.5 * dt, 1.0),
        tc6(False, False, 0.5 * dt, 2.0),
        tc6(False, False, dt, 2.0),
        tc6(False, True, dt / 6.0, 1.0),
    ]

    ys = [hidden]
    y = hidden
    kacc = hidden
    for _step in range(5):
        ybase = y
        ug_cur = ug
        for st in range(4):
            sa = spmm64_1(ug_cur, edges_r).reshape(2, n, 64)
            vg = tc5(sa, ug_cur, dinv64, bo1, Wo2)
            sb = spmm64_1(vg, edges_r).reshape(2, n, 64)
            kacc, yn, ug_cur = stages[st](sb, vg, dinv64, bo2, ybase, kacc, Wo1)
        y = yn
        ys.append(y)
        ug = ug_cur

    pred = _tc_call(
        _tc7_body,
        [nb6((64,)), _full((64, 1)), _full((1,))],
        nb1((6,)),
        jax.ShapeDtypeStruct((1, n, 6), F32),
    )(jnp.stack(ys), Wout, bout)
    return pred


# speedup vs baseline: 38.3590x; 1.4975x over previous
import functools

import jax
import jax.numpy as jnp
from jax import lax
from jax.experimental import pallas as pl
from jax.experimental.pallas import tpu as pltpu
from jax.experimental.pallas import tpu_sc as plsc

N_NODES = 10000
CHUNK = 80
NW = 32
F32 = jnp.float32


def _build_spmm(K: int, W: int, E: int):
    n = N_NODES
    epw = E // NW
    nchunk = epw // CHUNK
    rows_per_chunkrow = CHUNK
    cpw = epw // rows_per_chunkrow

    mesh = plsc.VectorSubcoreMesh(core_axis_name="c", subcore_axis_name="s",
                                  num_cores=2, num_subcores=16)

    def body(g_hbm, edges_hbm, out_hbm, src2, dst2, idxp, stage, zbuf, acc, sem, ssem):
        c = lax.axis_index("c")
        s = lax.axis_index("s")
        wid = s * 2 + c

        pltpu.sync_copy(edges_hbm.at[0, wid], src2)
        pltpu.sync_copy(edges_hbm.at[1, wid], dst2)

        def zrow(r, _):
            for i in range(W // 16):
                zbuf[r, pl.ds(i * 16, 16)] = jnp.zeros((16,), F32)
            return 0
        lax.fori_loop(0, 80, zrow, 0)

        def zero_stripe():
            @pl.when(s != 15)
            def _():
                def zm(m, _):
                    pltpu.sync_copy(zbuf, acc.at[pl.ds(s * 640 + m * 80, 80)])
                    return 0
                lax.fori_loop(0, 8, zm, 0)

            @pl.when(s == 15)
            def _():
                def zm(m, _):
                    pltpu.sync_copy(zbuf, acc.at[pl.ds(9600 + m * 80, 80)])
                    return 0
                lax.fori_loop(0, 5, zm, 0)

        zero_stripe()
        plsc.subcore_barrier()

        R = 4

        def fire(j, koff):
            slot = lax.rem(j, R)
            for i in range(CHUNK // 16):
                idxp[slot, pl.ds(i * 16, 16)] = src2[j, pl.ds(i * 16, 16)] + koff
            pltpu.make_async_copy(g_hbm.at[idxp.at[slot]], stage.at[slot], sem).start()

        def kbody(k, _):
            koff = k * n
            for j0 in range(R - 1):
                fire(j0, koff)

            def cbody(j, _):
                slot = lax.rem(j, R)

                @pl.when(j >= 1)
                def _():
                    sl1 = lax.rem(j - 1, R)
                    pltpu.make_async_copy(stage.at[sl1], acc.at[dst2.at[j - 1]],
                                          ssem).wait()

                @pl.when(j + R - 1 < nchunk)
                def _():
                    fire(j + R - 1, koff)

                pltpu.make_async_copy(g_hbm.at[idxp.at[slot]], stage.at[slot], sem).wait()
                pltpu.async_copy(stage.at[slot], acc.at[dst2.at[j]], ssem, add=True)
                return 0

            lax.fori_loop(0, nchunk, cbody, 0)
            pltpu.make_async_copy(stage.at[lax.rem(nchunk - 1, R)],
                                  acc.at[dst2.at[nchunk - 1]], ssem).wait()
            plsc.subcore_barrier()

            obase = (c * K + k) * n

            @pl.when(s != 15)
            def _():
                pltpu.sync_copy(acc.at[pl.ds(s * 640, 640)],
                                out_hbm.at[pl.ds(obase + s * 640, 640)])

            @pl.when(s == 15)
            def _():
                pltpu.sync_copy(acc.at[pl.ds(9600, 400)],
                                out_hbm.at[pl.ds(obase + 9600, 400)])

            zero_stripe()
            plsc.subcore_barrier()
            return 0

        lax.fori_loop(0, K, kbody, 0)

    kern = pl.kernel(
        body,
        out_type=jax.ShapeDtypeStruct((2 * K * n, W), F32),
        mesh=mesh,
        compiler_params=pltpu.CompilerParams(use_tc_tiling_on_sc=False),
        scratch_types=[
            pltpu.VMEM((cpw, CHUNK), jnp.int32),
            pltpu.VMEM((cpw, CHUNK), jnp.int32),
            pltpu.VMEM((4, CHUNK), jnp.int32),
            pltpu.VMEM((4, CHUNK, W), F32),
            pltpu.VMEM((80, W), F32),
            pltpu.VMEM_SHARED((n, W), F32),
            pltpu.SemaphoreType.DMA,
            pltpu.SemaphoreType.DMA,
        ],
    )
    return kern


BN = 1000


def _full(shape):
    return pl.BlockSpec(shape, lambda i: tuple(0 for _ in shape))


def _tc_call(body, in_specs, out_specs, out_shapes):
    return pl.pallas_call(
        body,
        grid=(N_NODES // BN,),
        in_specs=in_specs,
        out_specs=out_specs,
        out_shape=out_shapes,
    )


def _nb(*lead):
    def mk(trail):
        shape = tuple(lead) + (BN,) + tuple(trail)
        nlead = len(lead)
        def imap(i):
            return tuple(0 for _ in lead) + (i,) + tuple(0 for _ in trail)
        return pl.BlockSpec(shape, imap)
    return mk


def _tc1_body(deg2, xpad, dinv64, xg):
    deg = deg2[0, :, 0] + deg2[1, :, 0] + 1.0
    dinv = lax.rsqrt(deg)
    dinv64[...] = jnp.broadcast_to(dinv[:, None], dinv64.shape)
    xg[...] = xpad[...] * dinv[:, None]


def _tc2_body(s1, xg, dinv64, W1, b1, W2, zg2):
    dv = dinv64[...]
    Y = (s1[0] + s1[1] + xg[...]) * dv[:, :16]
    w1 = W1[...][0, :]
    for t in range(12):
        h1 = jnp.maximum(Y[:, t][:, None] * w1[None, :] + b1[...][None, :], 0.0)
        zg2[t] = jnp.dot(h1, W2[...], preferred_element_type=F32) * dv


def _tc3_body(sp, zg, dinv64, b, Wn, zgn):
    dv = dinv64[...]
    for t in range(12):
        h = jnp.maximum((sp[0, t] + sp[1, t] + zg[t]) * dv + b[...][None, :], 0.0)
        zgn[t] = jnp.dot(h, Wn[...], preferred_element_type=F32) * dv


def _tc4_body(sp, zg, dinv64, b3, Wa1, ba1, Wa2, ba2, WihT, bih, bhh, Wo1,
              hid_out, ug_out):
    dv = dinv64[...]
    hs = []
    ats = []
    for t in range(12):
        h = jnp.maximum((sp[0, t] + sp[1, t] + zg[t]) * dv + b3[...][None, :], 0.0)
        hs.append(h)
        a = jnp.tanh(jnp.dot(h, Wa1[...], preferred_element_type=F32) + ba1[...][None, :])
        ats.append(jnp.dot(a, Wa2[...], preferred_element_type=F32) + ba2[...][None, :])
    att = jnp.concatenate(ats, axis=1)
    m = jnp.max(att, axis=1, keepdims=True)
    e = jnp.exp(att - m)
    w = e / jnp.sum(e, axis=1, keepdims=True)
    nf = hs[0] * w[:, 0][:, None]
    for t in range(1, 12):
        nf = nf + hs[t] * w[:, t][:, None]
    gx = jnp.dot(nf, WihT[...], preferred_element_type=F32) + bih[...][None, :]
    bh = bhh[...]
    r = jax.nn.sigmoid(gx[:, :64] + bh[None, :64])
    z = jax.nn.sigmoid(gx[:, 64:128] + bh[None, 64:128])
    cand = jnp.tanh(gx[:, 128:] + r * bh[None, 128:])
    hidden = (1.0 - z) * cand
    hid_out[...] = hidden
    ug_out[...] = jnp.dot(hidden, Wo1[...], preferred_element_type=F32) * dv


def _tc5_body(sp, ug, dinv64, bo1, Wo2, vg):
    dv = dinv64[...]
    q = jnp.tanh((sp[0] + sp[1] + ug[...]) * dv + bo1[...][None, :])
    vg[...] = jnp.dot(q, Wo2[...], preferred_element_type=F32) * dv


def _tc6_body(sp, vg, dinv64, bo2, y, kacc, Wo1, kacc_out, ynext, ugnext,
              *, init, final, alpha, beta):
    dv = dinv64[...]
    kk = jnp.tanh((sp[0] + sp[1] + vg[...]) * dv + bo2[...][None, :])
    ka = kk if init else kacc[...] + beta * kk
    kacc_out[...] = ka
    yn = y[...] + (alpha * ka if final else alpha * kk)
    ynext[...] = yn
    ugnext[...] = jnp.dot(yn, Wo1[...], preferred_element_type=F32) * dv


def _tc7_body(ys, Wout, bout, out):
    cols = []
    for i in range(6):
        cols.append(jnp.dot(ys[i], Wout[...], preferred_element_type=F32)
                    + bout[...][None, :])
    out[...] = jnp.concatenate(cols, axis=1)[None]


def kernel(x, edge_index, W1, b1, W2, b2, W3, b3, Wa1, ba1, Wa2, ba2,
           W_ih, b_ih, W_hh, b_hh, Wo1, bo1, Wo2, bo2, Wout, bout):
    B, N, T = x.shape
    n = B * N
    E = edge_index.shape[1]
    edges_r = edge_index.reshape(2, NW, E // (NW * CHUNK), CHUNK)

    spmm16 = _build_spmm(1, 16, E)
    spmm64_12 = _build_spmm(12, 64, E)
    spmm64_1 = _build_spmm(1, 64, E)

    nb = _nb()
    nb2 = _nb(2)
    nb12 = _nb(12)
    nb2_12 = _nb(2, 12)
    nb6 = _nb(6)
    nb1 = _nb(1)

    ones16 = jnp.ones((n, 16), F32)
    deg2 = spmm16(ones16, edges_r).reshape(2, n, 16)

    xpad = jnp.concatenate([x.reshape(n, T), jnp.zeros((n, 16 - T), F32)], axis=1)

    dinv64, xg = _tc_call(
        _tc1_body,
        [nb2((16,)), nb((16,))],
        [nb((64,)), nb((16,))],
        [jax.ShapeDtypeStruct((n, 64), F32), jax.ShapeDtypeStruct((n, 16), F32)],
    )(deg2, xpad)

    s1 = spmm16(xg, edges_r).reshape(2, n, 16)

    zg2 = _tc_call(
        _tc2_body,
        [nb2((16,)), nb((16,)), nb((64,)), _full((1, 64)), _full((64,)), _full((64, 64))],
        nb12((64,)),
        jax.ShapeDtypeStruct((12, n, 64), F32),
    )(s1, xg, dinv64, W1, b1, W2)

    s2 = spmm64_12(zg2.reshape(12 * n, 64), edges_r).reshape(2, 12, n, 64)

    zg3 = _tc_call(
        _tc3_body,
        [nb2_12((64,)), nb12((64,)), nb((64,)), _full((64,)), _full((64, 64))],
        nb12((64,)),
        jax.ShapeDtypeStruct((12, n, 64), F32),
    )(s2, zg2, dinv64, b2, W3)

    s3 = spmm64_12(zg3.reshape(12 * n, 64), edges_r).reshape(2, 12, n, 64)

    hidden, ug = _tc_call(
        _tc4_body,
        [nb2_12((64,)), nb12((64,)), nb((64,)), _full((64,)),
         _full((64, 64)), _full((64,)), _full((64, 1)), _full((1,)),
         _full((64, 192)), _full((192,)), _full((192,)), _full((64, 64))],
        [nb((64,)), nb((64,))],
        [jax.ShapeDtypeStruct((n, 64), F32), jax.ShapeDtypeStruct((n, 64), F32)],
    )(s3, zg3, dinv64, b3, Wa1, ba1, Wa2, ba2, W_ih.T, b_ih, b_hh, Wo1)

    tc5 = _tc_call(
        _tc5_body,
        [nb2((64,)), nb((64,)), nb((64,)), _full((64,)), _full((64, 64))],
        nb((64,)),
        jax.ShapeDtypeStruct((n, 64), F32),
    )

    def tc6(init, final, alpha, beta):
        return _tc_call(
            functools.partial(_tc6_body, init=init, final=final,
                              alpha=alpha, beta=beta),
            [nb2((64,)), nb((64,)), nb((64,)), _full((64,)),
             nb((64,)), nb((64,)), _full((64, 64))],
            [nb((64,)), nb((64,)), nb((64,))],
            [jax.ShapeDtypeStruct((n, 64), F32)] * 3,
        )

    dt = 6.0 / 5.0
    stages = [
        tc6(True, False, 0.5 * dt, 1.0),
        tc6(False, False, 0.5 * dt, 2.0),
        tc6(False, False, dt, 2.0),
        tc6(False, True, dt / 6.0, 1.0),
    ]

    ys = [hidden]
    y = hidden
    kacc = hidden
    for _step in range(5):
        ybase = y
        ug_cur = ug
        for st in range(4):
            sa = spmm64_1(ug_cur, edges_r).reshape(2, n, 64)
            vg = tc5(sa, ug_cur, dinv64, bo1, Wo2)
            sb = spmm64_1(vg, edges_r).reshape(2, n, 64)
            kacc, yn, ug_cur = stages[st](sb, vg, dinv64, bo2, ybase, kacc, Wo1)
        y = yn
        ys.append(y)
        ug = ug_cur

    pred = _tc_call(
        _tc7_body,
        [nb6((64,)), _full((64, 1)), _full((1,))],
        nb1((6,)),
        jax.ShapeDtypeStruct((1, n, 6), F32),
    )(jnp.stack(ys), Wout, bout)
    return pred


# speedup vs baseline: 40.2381x; 1.0490x over previous
import functools

import jax
import jax.numpy as jnp
from jax import lax
from jax.experimental import pallas as pl
from jax.experimental.pallas import tpu as pltpu
from jax.experimental.pallas import tpu_sc as plsc

N_NODES = 10000
CHUNK = 80
NW = 32
F32 = jnp.float32


def _build_spmm(K: int, W: int, E: int):
    n = N_NODES
    epw = E // NW
    nchunk = epw // CHUNK
    rows_per_chunkrow = CHUNK
    cpw = epw // rows_per_chunkrow

    mesh = plsc.VectorSubcoreMesh(core_axis_name="c", subcore_axis_name="s",
                                  num_cores=2, num_subcores=16)

    def body(g_hbm, edges_hbm, out_hbm, src2, dst2, idxp, stage, zbuf, acc, sem, ssem):
        c = lax.axis_index("c")
        s = lax.axis_index("s")
        wid = s * 2 + c

        pltpu.sync_copy(edges_hbm.at[0, wid], src2)
        pltpu.sync_copy(edges_hbm.at[1, wid], dst2)

        def zrow(r, _):
            for i in range(W // 16):
                zbuf[r, pl.ds(i * 16, 16)] = jnp.zeros((16,), F32)
            return 0
        lax.fori_loop(0, 80, zrow, 0)

        def zero_stripe():
            @pl.when(s != 15)
            def _():
                def zm(m, _):
                    pltpu.sync_copy(zbuf, acc.at[pl.ds(s * 640 + m * 80, 80)])
                    return 0
                lax.fori_loop(0, 8, zm, 0)

            @pl.when(s == 15)
            def _():
                def zm(m, _):
                    pltpu.sync_copy(zbuf, acc.at[pl.ds(9600 + m * 80, 80)])
                    return 0
                lax.fori_loop(0, 5, zm, 0)

        zero_stripe()
        plsc.subcore_barrier()

        R = 8

        def fire(j, koff):
            slot = lax.rem(j, R)
            for i in range(CHUNK // 16):
                idxp[slot, pl.ds(i * 16, 16)] = src2[j, pl.ds(i * 16, 16)] + koff
            pltpu.make_async_copy(g_hbm.at[idxp.at[slot]], stage.at[slot], sem).start()

        def kbody(k, _):
            koff = k * n
            for j0 in range(R - 1):
                fire(j0, koff)

            def cbody(j, _):
                slot = lax.rem(j, R)

                @pl.when(j >= 1)
                def _():
                    sl1 = lax.rem(j - 1, R)
                    pltpu.make_async_copy(stage.at[sl1], acc.at[dst2.at[j - 1]],
                                          ssem).wait()

                @pl.when(j + R - 1 < nchunk)
                def _():
                    fire(j + R - 1, koff)

                pltpu.make_async_copy(g_hbm.at[idxp.at[slot]], stage.at[slot], sem).wait()
                pltpu.async_copy(stage.at[slot], acc.at[dst2.at[j]], ssem, add=True)
                return 0

            lax.fori_loop(0, nchunk, cbody, 0)
            pltpu.make_async_copy(stage.at[lax.rem(nchunk - 1, R)],
                                  acc.at[dst2.at[nchunk - 1]], ssem).wait()
            plsc.subcore_barrier()

            obase = (c * K + k) * n

            @pl.when(s != 15)
            def _():
                pltpu.sync_copy(acc.at[pl.ds(s * 640, 640)],
                                out_hbm.at[pl.ds(obase + s * 640, 640)])

            @pl.when(s == 15)
            def _():
                pltpu.sync_copy(acc.at[pl.ds(9600, 400)],
                                out_hbm.at[pl.ds(obase + 9600, 400)])

            zero_stripe()
            plsc.subcore_barrier()
            return 0

        lax.fori_loop(0, K, kbody, 0)

    kern = pl.kernel(
        body,
        out_type=jax.ShapeDtypeStruct((2 * K * n, W), F32),
        mesh=mesh,
        compiler_params=pltpu.CompilerParams(use_tc_tiling_on_sc=False),
        scratch_types=[
            pltpu.VMEM((cpw, CHUNK), jnp.int32),
            pltpu.VMEM((cpw, CHUNK), jnp.int32),
            pltpu.VMEM((8, CHUNK), jnp.int32),
            pltpu.VMEM((8, CHUNK, W), F32),
            pltpu.VMEM((80, W), F32),
            pltpu.VMEM_SHARED((n, W), F32),
            pltpu.SemaphoreType.DMA,
            pltpu.SemaphoreType.DMA,
        ],
    )
    return kern


def _full(shape):
    return pl.BlockSpec(shape, lambda i: tuple(0 for _ in shape))


def _tc_call(body, in_specs, out_specs, out_shapes, bn=1000):
    return pl.pallas_call(
        body,
        grid=(N_NODES // bn,),
        in_specs=in_specs,
        out_specs=out_specs,
        out_shape=out_shapes,
    )


def _nb(*lead, bn=1000):
    def mk(trail):
        shape = tuple(lead) + (bn,) + tuple(trail)
        def imap(i):
            return tuple(0 for _ in lead) + (i,) + tuple(0 for _ in trail)
        return pl.BlockSpec(shape, imap)
    return mk


def _tc1_body(deg2, xpad, dinv64, xg):
    deg = deg2[0, :, 0] + deg2[1, :, 0] + 1.0
    dinv = lax.rsqrt(deg)
    dinv64[...] = jnp.broadcast_to(dinv[:, None], dinv64.shape)
    xg[...] = xpad[...] * dinv[:, None]


def _tc2_body(s1, xg, dinv64, W1, b1, W2, zg2):
    dv = dinv64[...]
    Y = (s1[0] + s1[1] + xg[...]) * dv[:, :16]
    w1 = W1[...][0, :]
    for t in range(12):
        h1 = jnp.maximum(Y[:, t][:, None] * w1[None, :] + b1[...][None, :], 0.0)
        zg2[t] = jnp.dot(h1, W2[...], preferred_element_type=F32) * dv


def _tc3_body(sp, zg, dinv64, b, Wn, zgn):
    dv = dinv64[...]
    for t in range(12):
        h = jnp.maximum((sp[0, t] + sp[1, t] + zg[t]) * dv + b[...][None, :], 0.0)
        zgn[t] = jnp.dot(h, Wn[...], preferred_element_type=F32) * dv


def _tc4_body(sp, zg, dinv64, b3, Wa1, ba1, Wa2, ba2, WihT, bih, bhh, Wo1,
              hid_out, ug_out):
    dv = dinv64[...]
    hs = []
    ats = []
    for t in range(12):
        h = jnp.maximum((sp[0, t] + sp[1, t] + zg[t]) * dv + b3[...][None, :], 0.0)
        hs.append(h)
        a = jnp.tanh(jnp.dot(h, Wa1[...], preferred_element_type=F32) + ba1[...][None, :])
        ats.append(jnp.dot(a, Wa2[...], preferred_element_type=F32) + ba2[...][None, :])
    att = jnp.concatenate(ats, axis=1)
    m = jnp.max(att, axis=1, keepdims=True)
    e = jnp.exp(att - m)
    w = e / jnp.sum(e, axis=1, keepdims=True)
    nf = hs[0] * w[:, 0][:, None]
    for t in range(1, 12):
        nf = nf + hs[t] * w[:, t][:, None]
    gx = jnp.dot(nf, WihT[...], preferred_element_type=F32) + bih[...][None, :]
    bh = bhh[...]
    r = jax.nn.sigmoid(gx[:, :64] + bh[None, :64])
    z = jax.nn.sigmoid(gx[:, 64:128] + bh[None, 64:128])
    cand = jnp.tanh(gx[:, 128:] + r * bh[None, 128:])
    hidden = (1.0 - z) * cand
    hid_out[...] = hidden
    ug_out[...] = jnp.dot(hidden, Wo1[...], preferred_element_type=F32) * dv


def _tc5_body(sp, ug, dinv64, bo1, Wo2, vg):
    dv = dinv64[...]
    q = jnp.tanh((sp[0] + sp[1] + ug[...]) * dv + bo1[...][None, :])
    vg[...] = jnp.dot(q, Wo2[...], preferred_element_type=F32) * dv


def _tc6_body(sp, vg, dinv64, bo2, y, kacc, Wo1, kacc_out, ynext, ugnext,
              *, init, final, alpha, beta):
    dv = dinv64[...]
    kk = jnp.tanh((sp[0] + sp[1] + vg[...]) * dv + bo2[...][None, :])
    ka = kk if init else kacc[...] + beta * kk
    kacc_out[...] = ka
    yn = y[...] + (alpha * ka if final else alpha * kk)
    ynext[...] = yn
    ugnext[...] = jnp.dot(yn, Wo1[...], preferred_element_type=F32) * dv


def _tc7_body(ys, Wout, bout, out):
    cols = []
    for i in range(6):
        cols.append(jnp.dot(ys[i], Wout[...], preferred_element_type=F32)
                    + bout[...][None, :])
    out[...] = jnp.concatenate(cols, axis=1)[None]


def kernel(x, edge_index, W1, b1, W2, b2, W3, b3, Wa1, ba1, Wa2, ba2,
           W_ih, b_ih, W_hh, b_hh, Wo1, bo1, Wo2, bo2, Wout, bout):
    B, N, T = x.shape
    n = B * N
    E = edge_index.shape[1]
    edges_r = edge_index.reshape(2, NW, E // (NW * CHUNK), CHUNK)

    spmm16 = _build_spmm(1, 16, E)
    spmm64_12 = _build_spmm(12, 64, E)
    spmm64_1 = _build_spmm(1, 64, E)

    nb = _nb()
    nb2 = _nb(2)
    nb12 = _nb(12)
    nb2_12 = _nb(2, 12)
    nb6 = _nb(6)
    nb1 = _nb(1)

    ones16 = jnp.ones((n, 16), F32)
    deg2 = spmm16(ones16, edges_r).reshape(2, n, 16)

    xpad = jnp.concatenate([x.reshape(n, T), jnp.zeros((n, 16 - T), F32)], axis=1)

    dinv64, xg = _tc_call(
        _tc1_body,
        [nb2((16,)), nb((16,))],
        [nb((64,)), nb((16,))],
        [jax.ShapeDtypeStruct((n, 64), F32), jax.ShapeDtypeStruct((n, 16), F32)],
    )(deg2, xpad)

    s1 = spmm16(xg, edges_r).reshape(2, n, 16)

    zg2 = _tc_call(
        _tc2_body,
        [nb2((16,)), nb((16,)), nb((64,)), _full((1, 64)), _full((64,)), _full((64, 64))],
        nb12((64,)),
        jax.ShapeDtypeStruct((12, n, 64), F32),
    )(s1, xg, dinv64, W1, b1, W2)

    s2 = spmm64_12(zg2.reshape(12 * n, 64), edges_r).reshape(2, 12, n, 64)

    zg3 = _tc_call(
        _tc3_body,
        [nb2_12((64,)), nb12((64,)), nb((64,)), _full((64,)), _full((64, 64))],
        nb12((64,)),
        jax.ShapeDtypeStruct((12, n, 64), F32),
    )(s2, zg2, dinv64, b2, W3)

    s3 = spmm64_12(zg3.reshape(12 * n, 64), edges_r).reshape(2, 12, n, 64)

    hidden, ug = _tc_call(
        _tc4_body,
        [nb2_12((64,)), nb12((64,)), nb((64,)), _full((64,)),
         _full((64, 64)), _full((64,)), _full((64, 1)), _full((1,)),
         _full((64, 192)), _full((192,)), _full((192,)), _full((64, 64))],
        [nb((64,)), nb((64,))],
        [jax.ShapeDtypeStruct((n, 64), F32), jax.ShapeDtypeStruct((n, 64), F32)],
    )(s3, zg3, dinv64, b3, Wa1, ba1, Wa2, ba2, W_ih.T, b_ih, b_hh, Wo1)

    obn = 2000
    onb = _nb(bn=obn)((64,))
    onb2 = _nb(2, bn=obn)((64,))
    tc5 = _tc_call(
        _tc5_body,
        [onb2, onb, onb, _full((64,)), _full((64, 64))],
        onb,
        jax.ShapeDtypeStruct((n, 64), F32),
        bn=obn,
    )

    def tc6(init, final, alpha, beta):
        return _tc_call(
            functools.partial(_tc6_body, init=init, final=final,
                              alpha=alpha, beta=beta),
            [onb2, onb, onb, _full((64,)), onb, onb, _full((64, 64))],
            [onb, onb, onb],
            [jax.ShapeDtypeStruct((n, 64), F32)] * 3,
            bn=obn,
        )

    dt = 6.0 / 5.0
    stages = [
        tc6(True, False, 0.5 * dt, 1.0),
        tc6(False, False, 0.5 * dt, 2.0),
        tc6(False, False, dt, 2.0),
        tc6(False, True, dt / 6.0, 1.0),
    ]

    ys = [hidden]
    y = hidden
    kacc = hidden
    for _step in range(5):
        ybase = y
        ug_cur = ug
        for st in range(4):
            sa = spmm64_1(ug_cur, edges_r).reshape(2, n, 64)
            vg = tc5(sa, ug_cur, dinv64, bo1, Wo2)
            sb = spmm64_1(vg, edges_r).reshape(2, n, 64)
            kacc, yn, ug_cur = stages[st](sb, vg, dinv64, bo2, ybase, kacc, Wo1)
        y = yn
        ys.append(y)
        ug = ug_cur

    pred = _tc_call(
        _tc7_body,
        [nb6((64,)), _full((64, 1)), _full((1,))],
        nb1((6,)),
        jax.ShapeDtypeStruct((1, n, 6), F32),
    )(jnp.stack(ys), Wout, bout)
    return pred


# speedup vs baseline: 44.0675x; 1.0952x over previous
import functools

import jax
import jax.numpy as jnp
from jax import lax
from jax.experimental import pallas as pl
from jax.experimental.pallas import tpu as pltpu
from jax.experimental.pallas import tpu_sc as plsc

N_NODES = 10000
CHUNK = 80
NW = 32
F32 = jnp.float32


def _build_spmm(K: int, W: int, E: int, dt=F32):
    nlane = 32 if dt == jnp.bfloat16 else 16
    n = N_NODES
    epw = E // NW
    nchunk = epw // CHUNK
    rows_per_chunkrow = CHUNK
    cpw = epw // rows_per_chunkrow

    mesh = plsc.VectorSubcoreMesh(core_axis_name="c", subcore_axis_name="s",
                                  num_cores=2, num_subcores=16)

    def body(g_hbm, edges_hbm, out_hbm, src2, dst2, idxp, stage, zbuf, acc, sem, ssem):
        c = lax.axis_index("c")
        s = lax.axis_index("s")
        wid = s * 2 + c

        pltpu.sync_copy(edges_hbm.at[0, wid], src2)
        pltpu.sync_copy(edges_hbm.at[1, wid], dst2)

        def zrow(r, _):
            for i in range(W // nlane):
                zbuf[r, pl.ds(i * nlane, nlane)] = jnp.zeros((nlane,), dt)
            return 0
        lax.fori_loop(0, 80, zrow, 0)

        def zero_stripe():
            @pl.when(s != 15)
            def _():
                def zm(m, _):
                    pltpu.sync_copy(zbuf, acc.at[pl.ds(s * 640 + m * 80, 80)])
                    return 0
                lax.fori_loop(0, 8, zm, 0)

            @pl.when(s == 15)
            def _():
                def zm(m, _):
                    pltpu.sync_copy(zbuf, acc.at[pl.ds(9600 + m * 80, 80)])
                    return 0
                lax.fori_loop(0, 5, zm, 0)

        zero_stripe()
        plsc.subcore_barrier()

        R = 8

        def fire(j, koff):
            slot = lax.rem(j, R)
            for i in range(CHUNK // 16):
                idxp[slot, pl.ds(i * 16, 16)] = src2[j, pl.ds(i * 16, 16)] + koff
            pltpu.make_async_copy(g_hbm.at[idxp.at[slot]], stage.at[slot], sem).start()

        def kbody(k, _):
            koff = k * n
            for j0 in range(R - 1):
                fire(j0, koff)

            def cbody(j, _):
                slot = lax.rem(j, R)

                @pl.when(j >= 1)
                def _():
                    sl1 = lax.rem(j - 1, R)
                    pltpu.make_async_copy(stage.at[sl1], acc.at[dst2.at[j - 1]],
                                          ssem).wait()

                @pl.when(j + R - 1 < nchunk)
                def _():
                    fire(j + R - 1, koff)

                pltpu.make_async_copy(g_hbm.at[idxp.at[slot]], stage.at[slot], sem).wait()
                pltpu.async_copy(stage.at[slot], acc.at[dst2.at[j]], ssem, add=True)
                return 0

            lax.fori_loop(0, nchunk, cbody, 0)
            pltpu.make_async_copy(stage.at[lax.rem(nchunk - 1, R)],
                                  acc.at[dst2.at[nchunk - 1]], ssem).wait()
            plsc.subcore_barrier()

            obase = (c * K + k) * n

            @pl.when(s != 15)
            def _():
                pltpu.sync_copy(acc.at[pl.ds(s * 640, 640)],
                                out_hbm.at[pl.ds(obase + s * 640, 640)])

            @pl.when(s == 15)
            def _():
                pltpu.sync_copy(acc.at[pl.ds(9600, 400)],
                                out_hbm.at[pl.ds(obase + 9600, 400)])

            zero_stripe()
            plsc.subcore_barrier()
            return 0

        lax.fori_loop(0, K, kbody, 0)

    kern = pl.kernel(
        body,
        out_type=jax.ShapeDtypeStruct((2 * K * n, W), dt),
        mesh=mesh,
        compiler_params=pltpu.CompilerParams(use_tc_tiling_on_sc=False),
        scratch_types=[
            pltpu.VMEM((cpw, CHUNK), jnp.int32),
            pltpu.VMEM((cpw, CHUNK), jnp.int32),
            pltpu.VMEM((8, CHUNK), jnp.int32),
            pltpu.VMEM((8, CHUNK, W), dt),
            pltpu.VMEM((80, W), dt),
            pltpu.VMEM_SHARED((n, W), dt),
            pltpu.SemaphoreType.DMA,
            pltpu.SemaphoreType.DMA,
        ],
    )
    return kern


def _full(shape):
    return pl.BlockSpec(shape, lambda i: tuple(0 for _ in shape))


def _tc_call(body, in_specs, out_specs, out_shapes, bn=1000):
    return pl.pallas_call(
        body,
        grid=(N_NODES // bn,),
        in_specs=in_specs,
        out_specs=out_specs,
        out_shape=out_shapes,
    )


def _nb(*lead, bn=1000):
    def mk(trail):
        shape = tuple(lead) + (bn,) + tuple(trail)
        def imap(i):
            return tuple(0 for _ in lead) + (i,) + tuple(0 for _ in trail)
        return pl.BlockSpec(shape, imap)
    return mk


def _tc1_body(deg2, xpad, dinv64, xg):
    deg = deg2[0, :, 0] + deg2[1, :, 0] + 1.0
    dinv = lax.rsqrt(deg)
    dinv64[...] = jnp.broadcast_to(dinv[:, None], dinv64.shape)
    xg[...] = xpad[...] * dinv[:, None]


def _tc2_body(s1, xg, dinv64, W1, b1, W2, zg2):
    dv = dinv64[...]
    Y = (s1[0] + s1[1] + xg[...]) * dv[:, :16]
    w1 = W1[...][0, :]
    for t in range(12):
        h1 = jnp.maximum(Y[:, t][:, None] * w1[None, :] + b1[...][None, :], 0.0)
        zg2[t] = (jnp.dot(h1, W2[...], preferred_element_type=F32) * dv
                  ).astype(zg2.dtype)


def _tc3_body(sp, zg, dinv64, b, Wn, zgn):
    dv = dinv64[...]
    for t in range(12):
        s = sp[0, t].astype(F32) + sp[1, t].astype(F32) + zg[t].astype(F32)
        h = jnp.maximum(s * dv + b[...][None, :], 0.0)
        zgn[t] = (jnp.dot(h, Wn[...], preferred_element_type=F32) * dv
                  ).astype(zgn.dtype)


def _tc4_body(sp, zg, dinv64, b3, Wa1, ba1, Wa2, ba2, WihT, bih, bhh, Wo1,
              hid_out, ug_out):
    dv = dinv64[...]
    hs = []
    ats = []
    for t in range(12):
        s = sp[0, t].astype(F32) + sp[1, t].astype(F32) + zg[t].astype(F32)
        h = jnp.maximum(s * dv + b3[...][None, :], 0.0)
        hs.append(h)
        a = jnp.tanh(jnp.dot(h, Wa1[...], preferred_element_type=F32) + ba1[...][None, :])
        ats.append(jnp.dot(a, Wa2[...], preferred_element_type=F32) + ba2[...][None, :])
    att = jnp.concatenate(ats, axis=1)
    m = jnp.max(att, axis=1, keepdims=True)
    e = jnp.exp(att - m)
    w = e / jnp.sum(e, axis=1, keepdims=True)
    nf = hs[0] * w[:, 0][:, None]
    for t in range(1, 12):
        nf = nf + hs[t] * w[:, t][:, None]
    gx = jnp.dot(nf, WihT[...], preferred_element_type=F32) + bih[...][None, :]
    bh = bhh[...]
    r = jax.nn.sigmoid(gx[:, :64] + bh[None, :64])
    z = jax.nn.sigmoid(gx[:, 64:128] + bh[None, 64:128])
    cand = jnp.tanh(gx[:, 128:] + r * bh[None, 128:])
    hidden = (1.0 - z) * cand
    hid_out[...] = hidden
    ug_out[...] = (jnp.dot(hidden, Wo1[...], preferred_element_type=F32) * dv
                   ).astype(ug_out.dtype)


def _tc5_body(sp, ug, dinv64, bo1, Wo2, vg):
    dv = dinv64[...]
    s = sp[0].astype(F32) + sp[1].astype(F32) + ug[...].astype(F32)
    q = jnp.tanh(s * dv + bo1[...][None, :])
    vg[...] = (jnp.dot(q, Wo2[...], preferred_element_type=F32) * dv
               ).astype(vg.dtype)


def _tc6_body(sp, vg, dinv64, bo2, y, kacc, Wo1, kacc_out, ynext, ugnext,
              *, init, final, alpha, beta):
    dv = dinv64[...]
    s = sp[0].astype(F32) + sp[1].astype(F32) + vg[...].astype(F32)
    kk = jnp.tanh(s * dv + bo2[...][None, :])
    ka = kk if init else kacc[...] + beta * kk
    kacc_out[...] = ka
    yn = y[...] + (alpha * ka if final else alpha * kk)
    ynext[...] = yn
    ugnext[...] = (jnp.dot(yn, Wo1[...], preferred_element_type=F32) * dv
                   ).astype(ugnext.dtype)


def _tc7_body(ys, Wout, bout, out):
    cols = []
    for i in range(6):
        cols.append(jnp.dot(ys[i], Wout[...], preferred_element_type=F32)
                    + bout[...][None, :])
    out[...] = jnp.concatenate(cols, axis=1)[None]


def kernel(x, edge_index, W1, b1, W2, b2, W3, b3, Wa1, ba1, Wa2, ba2,
           W_ih, b_ih, W_hh, b_hh, Wo1, bo1, Wo2, bo2, Wout, bout):
    B, N, T = x.shape
    n = B * N
    E = edge_index.shape[1]
    edges_r = edge_index.reshape(2, NW, E // (NW * CHUNK), CHUNK)

    BF16 = jnp.bfloat16
    spmm16 = _build_spmm(1, 16, E)
    spmm64_12 = _build_spmm(12, 64, E, BF16)
    spmm64_1 = _build_spmm(1, 64, E)

    nb = _nb()
    nb2 = _nb(2)
    nb12 = _nb(12)
    nb2_12 = _nb(2, 12)
    nb6 = _nb(6)
    nb1 = _nb(1)

    ones16 = jnp.ones((n, 16), F32)
    deg2 = spmm16(ones16, edges_r).reshape(2, n, 16)

    xpad = jnp.concatenate([x.reshape(n, T), jnp.zeros((n, 16 - T), F32)], axis=1)

    dinv64, xg = _tc_call(
        _tc1_body,
        [nb2((16,)), nb((16,))],
        [nb((64,)), nb((16,))],
        [jax.ShapeDtypeStruct((n, 64), F32), jax.ShapeDtypeStruct((n, 16), F32)],
    )(deg2, xpad)

    s1 = spmm16(xg, edges_r).reshape(2, n, 16)

    zg2 = _tc_call(
        _tc2_body,
        [nb2((16,)), nb((16,)), nb((64,)), _full((1, 64)), _full((64,)), _full((64, 64))],
        nb12((64,)),
        jax.ShapeDtypeStruct((12, n, 64), BF16),
    )(s1, xg, dinv64, W1, b1, W2)

    s2 = spmm64_12(zg2.reshape(12 * n, 64), edges_r).reshape(2, 12, n, 64)

    zg3 = _tc_call(
        _tc3_body,
        [nb2_12((64,)), nb12((64,)), nb((64,)), _full((64,)), _full((64, 64))],
        nb12((64,)),
        jax.ShapeDtypeStruct((12, n, 64), BF16),
    )(s2, zg2, dinv64, b2, W3)

    s3 = spmm64_12(zg3.reshape(12 * n, 64), edges_r).reshape(2, 12, n, 64)

    hidden, ug = _tc_call(
        _tc4_body,
        [nb2_12((64,)), nb12((64,)), nb((64,)), _full((64,)),
         _full((64, 64)), _full((64,)), _full((64, 1)), _full((1,)),
         _full((64, 192)), _full((192,)), _full((192,)), _full((64, 64))],
        [nb((64,)), nb((64,))],
        [jax.ShapeDtypeStruct((n, 64), F32), jax.ShapeDtypeStruct((n, 64), F32)],
    )(s3, zg3, dinv64, b3, Wa1, ba1, Wa2, ba2, W_ih.T, b_ih, b_hh, Wo1)

    obn = 2000
    onb = _nb(bn=obn)((64,))
    onb2 = _nb(2, bn=obn)((64,))
    tc5 = _tc_call(
        _tc5_body,
        [onb2, onb, onb, _full((64,)), _full((64, 64))],
        onb,
        jax.ShapeDtypeStruct((n, 64), F32),
        bn=obn,
    )

    def tc6(init, final, alpha, beta):
        return _tc_call(
            functools.partial(_tc6_body, init=init, final=final,
                              alpha=alpha, beta=beta),
            [onb2, onb, onb, _full((64,)), onb, onb, _full((64, 64))],
            [onb, onb, onb],
            [jax.ShapeDtypeStruct((n, 64), F32)] * 3,
            bn=obn,
        )

    dt = 6.0 / 5.0
    stages = [
        tc6(True, False, 0.5 * dt, 1.0),
        tc6(False, False, 0.5 * dt, 2.0),
        tc6(False, False, dt, 2.0),
        tc6(False, True, dt / 6.0, 1.0),
    ]

    ys = [hidden]
    y = hidden
    kacc = hidden
    for _step in range(5):
        ybase = y
        ug_cur = ug
        for st in range(4):
            sa = spmm64_1(ug_cur, edges_r).reshape(2, n, 64)
            vg = tc5(sa, ug_cur, dinv64, bo1, Wo2)
            sb = spmm64_1(vg, edges_r).reshape(2, n, 64)
            kacc, yn, ug_cur = stages[st](sb, vg, dinv64, bo2, ybase, kacc, Wo1)
        y = yn
        ys.append(y)
        ug = ug_cur

    pred = _tc_call(
        _tc7_body,
        [nb6((64,)), _full((64, 1)), _full((1,))],
        nb1((6,)),
        jax.ShapeDtypeStruct((1, n, 6), F32),
    )(jnp.stack(ys), Wout, bout)
    return pred
